# baseline (speedup 1.0000x reference)
import functools

import jax
import jax.numpy as jnp
from jax import lax
from jax.experimental import pallas as pl
from jax.experimental.pallas import tpu as pltpu
from jax.experimental.pallas import tpu_sc as plsc

VOCAB = 1000000
EMB = 64
HID = 1024
OUT = 128
NTOK = 16384

NW = 32
BPW = NTOK // NW
CHUNK = 32
NCH = BPW // CHUNK
LANES = 16
NVEC = EMB // LANES


def _sc_partial_sums(idx, table_flat):
    mesh = plsc.VectorSubcoreMesh(core_axis_name="c", subcore_axis_name="s")

    @functools.partial(
        pl.kernel,
        mesh=mesh,
        out_type=jax.ShapeDtypeStruct((NW * EMB,), jnp.float32),
        compiler_params=pltpu.CompilerParams(use_tc_tiling_on_sc=False),
        scratch_types=[
            pltpu.VMEM((BPW,), jnp.int32),
            pltpu.VMEM((2, CHUNK * EMB), jnp.float32),
            pltpu.VMEM((EMB,), jnp.float32),
            pltpu.SemaphoreType.DMA,
            pltpu.SemaphoreType.DMA,
        ],
    )
    def k(idx_hbm, table_hbm, out_hbm, idx_v, buf_v, acc_v, sem0, sem1):
        wid = lax.axis_index("s") * 2 + lax.axis_index("c")
        base = pl.multiple_of(wid * BPW, BPW)
        pltpu.sync_copy(idx_hbm.at[pl.ds(base, BPW)], idx_v)
        sems = (sem0, sem1)

        def fire(c, slot, sem):
            for g in range(CHUNK // LANES):
                v = idx_v[pl.ds(c * CHUNK + g * LANES, LANES)]
                voff = v * EMB
                for l in range(LANES):
                    pltpu.async_copy(
                        table_hbm.at[pl.ds(pl.multiple_of(voff[l], EMB), EMB)],
                        buf_v.at[slot, pl.ds((g * LANES + l) * EMB, EMB)],
                        sem,
                    )

        def drain(slot, sem):
            pltpu.make_async_copy(
                table_hbm.at[pl.ds(0, CHUNK * EMB)], buf_v.at[slot], sem
            ).wait()

        def accum(slot, acc):
            def ab(j, carry):
                return tuple(
                    carry[v] + buf_v[slot, pl.ds(j * EMB + v * LANES, LANES)]
                    for v in range(NVEC)
                )
            return lax.fori_loop(0, CHUNK, ab, acc)

        zero = jnp.zeros((LANES,), jnp.float32)
        acc = (zero,) * NVEC
        fire(0, 0, sems[0])
        for c in range(NCH):
            slot = c % 2
            if c + 1 < NCH:
                fire(c + 1, (c + 1) % 2, sems[(c + 1) % 2])
            drain(slot, sems[slot])
            acc = accum(slot, acc)
        for v in range(NVEC):
            acc_v[pl.ds(v * LANES, LANES)] = acc[v]
        pltpu.sync_copy(acc_v, out_hbm.at[pl.ds(pl.multiple_of(wid * EMB, EMB), EMB)])

    return k(idx, table_flat)


def _tc_ffnn(partials, W1, b1, W2, b2):
    def body(p_ref, w1_ref, b1_ref, w2_ref, b2_ref, o_ref):
        bag = jnp.sum(p_ref[...], axis=0, keepdims=True) * (1.0 / NTOK)
        h = jnp.dot(bag, w1_ref[...], preferred_element_type=jnp.float32)
        h = jnp.maximum(h + b1_ref[...], 0.0)
        logits = jnp.dot(h, w2_ref[...], preferred_element_type=jnp.float32)
        logits = logits + b2_ref[...]
        m = jnp.max(logits, axis=-1, keepdims=True)
        lse = jnp.log(jnp.sum(jnp.exp(logits - m), axis=-1, keepdims=True)) + m
        o_ref[...] = logits - lse

    return pl.pallas_call(
        body,
        out_shape=jax.ShapeDtypeStruct((1, OUT), jnp.float32),
    )(partials, W1, b1.reshape(1, HID), W2, b2.reshape(1, OUT))


def kernel(input, embeddings, W1, b1, W2, b2):
    partials = _sc_partial_sums(input, embeddings.reshape(-1))
    return _tc_ffnn(partials.reshape(NW, EMB), W1, b1, W2, b2)


# speedup vs baseline: 1.4440x; 1.4440x over previous
import functools

import jax
import jax.numpy as jnp
from jax import lax
from jax.experimental import pallas as pl
from jax.experimental.pallas import tpu as pltpu
from jax.experimental.pallas import tpu_sc as plsc

VOCAB = 1000000
EMB = 64
HID = 1024
OUT = 128
NTOK = 16384

NW = 32
LANES = 16

BBLK = 1664
NBLK = 601
CROWS = 608
CBIG = CROWS * BBLK
CPT = CBIG // 16
ZCH = CPT // 16


def _sc_counts(idx128):
    mesh = plsc.VectorSubcoreMesh(core_axis_name="c", subcore_axis_name="s")

    @functools.partial(
        pl.kernel,
        mesh=mesh,
        out_type=(
            jax.ShapeDtypeStruct((CBIG,), jnp.float32),
            jax.ShapeDtypeStruct((CBIG,), jnp.float32),
        ),
        scratch_types=[
            pltpu.VMEM((4, 128), jnp.int32),
            pltpu.VMEM((128,), jnp.float32),
            pltpu.VMEM((ZCH,), jnp.float32),
            pltpu.VMEM_SHARED((CBIG,), jnp.float32),
        ],
    )
    def k(idx_hbm, out0_hbm, out1_hbm, idx_v, ones_v, zeros_v, shared):
        s = lax.axis_index("s")
        c = lax.axis_index("c")
        g = s * 2 + c
        pltpu.sync_copy(idx_hbm.at[pl.ds(g * 4, 4)], idx_v)

        zero = jnp.zeros((LANES,), jnp.float32)

        def zv(r, carry):
            zeros_v[pl.ds(pl.multiple_of(r * LANES, LANES), LANES)] = zero
            return carry
        lax.fori_loop(0, ZCH // LANES, zv, 0)
        for l in range(8):
            ones_v[pl.ds(l * LANES, LANES)] = jnp.ones((LANES,), jnp.float32)

        base = pl.multiple_of(s * CPT, 8)
        for kk in range(16):
            pltpu.sync_copy(
                zeros_v, shared.at[pl.ds(base + kk * ZCH, ZCH)]
            )
        plsc.subcore_barrier()

        for j in range(4):
            pltpu.sync_copy(ones_v, shared.at[idx_v.at[j]], add=True)
        plsc.subcore_barrier()

        out = [out0_hbm, out1_hbm]
        for ci in range(2):
            @pl.when(c == ci)
            def _():
                pltpu.sync_copy(
                    shared.at[pl.ds(base, CPT)],
                    out[ci].at[pl.ds(base, CPT)],
                )

    return k(idx128)


def _tc_matvec(table_t, c0, c1):
    def body(t_ref, c0_ref, c1_ref, o_ref):
        i = pl.program_id(0)
        cc = c0_ref[pl.ds(i, 1), :] + c1_ref[pl.ds(i, 1), :]
        tb = t_ref[...]

        def masked(t):
            col = i * BBLK + jax.lax.broadcasted_iota(jnp.int32, (1, BBLK), 1)
            return jnp.where(col < VOCAB, t, 0.0)

        tb = lax.cond(i == NBLK - 1, masked, lambda t: t, tb)
        part = lax.dot_general(
            cc, tb, (((1,), (1,)), ((), ())),
            preferred_element_type=jnp.float32,
        )

        @pl.when(i == 0)
        def _():
            o_ref[...] = jnp.zeros_like(o_ref)

        o_ref[...] += part

    return pl.pallas_call(
        body,
        grid=(NBLK,),
        in_specs=[
            pl.BlockSpec((EMB, BBLK), lambda i: (0, i)),
            pl.BlockSpec((CROWS, BBLK), lambda i: (0, 0)),
            pl.BlockSpec((CROWS, BBLK), lambda i: (0, 0)),
        ],
        out_specs=pl.BlockSpec((1, EMB), lambda i: (0, 0)),
        out_shape=jax.ShapeDtypeStruct((1, EMB), jnp.float32),
    )(table_t, c0, c1)


def _tc_ffnn(bag_sum, W1, b1, W2, b2):
    def body(p_ref, w1_ref, b1_ref, w2_ref, b2_ref, o_ref):
        bag = p_ref[...] * (1.0 / NTOK)
        h = jnp.dot(bag, w1_ref[...], preferred_element_type=jnp.float32)
        h = jnp.maximum(h + b1_ref[...], 0.0)
        logits = jnp.dot(h, w2_ref[...], preferred_element_type=jnp.float32)
        logits = logits + b2_ref[...]
        m = jnp.max(logits, axis=-1, keepdims=True)
        lse = jnp.log(jnp.sum(jnp.exp(logits - m), axis=-1, keepdims=True)) + m
        o_ref[...] = logits - lse

    return pl.pallas_call(
        body,
        out_shape=jax.ShapeDtypeStruct((1, OUT), jnp.float32),
    )(bag_sum, W1, b1.reshape(1, HID), W2, b2.reshape(1, OUT))


def kernel(input, embeddings, W1, b1, W2, b2):
    c0, c1 = _sc_counts(input.reshape(128, 128))
    bag_sum = _tc_matvec(
        embeddings.T, c0.reshape(CROWS, BBLK), c1.reshape(CROWS, BBLK)
    )
    return _tc_ffnn(bag_sum, W1, b1, W2, b2)


# speedup vs baseline: 4.6280x; 3.2050x over previous
import functools

import jax
import jax.numpy as jnp
from jax import lax
from jax.experimental import pallas as pl
from jax.experimental.pallas import tpu as pltpu
from jax.experimental.pallas import tpu_sc as plsc

VOCAB = 1000000
EMB = 64
HID = 1024
OUT = 128
NTOK = 16384

LANES = 16

CW = 1664
CROWS = 608
CBIG = CROWS * CW
CPT = CBIG // 16
ZCH = CPT // 16

RPB = 8
MBLK = RPB * CW
NMAIN = 75
TAILC = NMAIN * RPB


def _sc_counts(idx128):
    mesh = plsc.VectorSubcoreMesh(core_axis_name="c", subcore_axis_name="s")

    @functools.partial(
        pl.kernel,
        mesh=mesh,
        out_type=(
            jax.ShapeDtypeStruct((CBIG,), jnp.float32),
            jax.ShapeDtypeStruct((CBIG,), jnp.float32),
        ),
        scratch_types=[
            pltpu.VMEM((4, 128), jnp.int32),
            pltpu.VMEM((128,), jnp.float32),
            pltpu.VMEM((ZCH,), jnp.float32),
            pltpu.VMEM_SHARED((CBIG,), jnp.float32),
        ],
    )
    def k(idx_hbm, out0_hbm, out1_hbm, idx_v, ones_v, zeros_v, shared):
        s = lax.axis_index("s")
        c = lax.axis_index("c")
        g = s * 2 + c
        pltpu.sync_copy(idx_hbm.at[pl.ds(g * 4, 4)], idx_v)

        zero = jnp.zeros((LANES,), jnp.float32)

        def zv(r, carry):
            zeros_v[pl.ds(pl.multiple_of(r * LANES, LANES), LANES)] = zero
            return carry
        lax.fori_loop(0, ZCH // LANES, zv, 0)
        for l in range(8):
            ones_v[pl.ds(l * LANES, LANES)] = jnp.ones((LANES,), jnp.float32)

        base = pl.multiple_of(s * CPT, 8)
        for kk in range(16):
            pltpu.sync_copy(
                zeros_v, shared.at[pl.ds(base + kk * ZCH, ZCH)]
            )
        plsc.subcore_barrier()

        for j in range(4):
            pltpu.sync_copy(ones_v, shared.at[idx_v.at[j]], add=True)
        plsc.subcore_barrier()

        out = [out0_hbm, out1_hbm]
        for ci in range(2):
            @pl.when(c == ci)
            def _():
                pltpu.sync_copy(
                    shared.at[pl.ds(base, CPT)],
                    out[ci].at[pl.ds(base, CPT)],
                )

    return k(idx128)


def _tc_matvec_main(table_t, c0, c1):
    def body(t_ref, c0_ref, c1_ref, o_ref):
        cc = c0_ref[...] + c1_ref[...]
        tb = t_ref[...]
        acc = jnp.zeros((EMB, 128), jnp.float32)
        for r in range(RPB):
            prod = tb[:, r * CW:(r + 1) * CW] * cc[r:r + 1, :]
            for gg in range(CW // 128):
                acc = acc + prod[:, gg * 128:(gg + 1) * 128]

        @pl.when(pl.program_id(0) == 0)
        def _():
            o_ref[...] = jnp.zeros_like(o_ref)

        o_ref[...] += acc

    return pl.pallas_call(
        body,
        grid=(NMAIN,),
        in_specs=[
            pl.BlockSpec((EMB, MBLK), lambda i: (0, i)),
            pl.BlockSpec((RPB, CW), lambda i: (i, 0)),
            pl.BlockSpec((RPB, CW), lambda i: (i, 0)),
        ],
        out_specs=pl.BlockSpec((EMB, 128), lambda i: (0, 0)),
        out_shape=jax.ShapeDtypeStruct((EMB, 128), jnp.float32),
    )(table_t, c0, c1)


def _tc_matvec_tail(table_t, c0, c1):
    def body(t_ref, c0_ref, c1_ref, o_ref):
        cc = c0_ref[pl.ds(TAILC % 8, 1), :] + c1_ref[pl.ds(TAILC % 8, 1), :]
        tb = t_ref[...]
        acc = jnp.zeros((EMB, 128), jnp.float32)
        for gg in range(CW // 128):
            prod = tb[:, gg * 128:(gg + 1) * 128] * cc[:, gg * 128:(gg + 1) * 128]
            if (gg + 1) * 128 > VOCAB - NMAIN * MBLK:
                valid = jax.lax.broadcasted_iota(jnp.int32, (1, 128), 1) < (
                    VOCAB - NMAIN * MBLK - gg * 128
                )
                prod = jnp.where(valid, prod, 0.0)
            acc = acc + prod
        o_ref[...] = acc

    return pl.pallas_call(
        body,
        grid=(1,),
        in_specs=[
            pl.BlockSpec((EMB, CW), lambda i: (0, NMAIN * RPB)),
            pl.BlockSpec((8, CW), lambda i: (TAILC // 8, 0)),
            pl.BlockSpec((8, CW), lambda i: (TAILC // 8, 0)),
        ],
        out_specs=pl.BlockSpec((EMB, 128), lambda i: (0, 0)),
        out_shape=jax.ShapeDtypeStruct((EMB, 128), jnp.float32),
    )(table_t, c0, c1)


def _tc_ffnn(acc_main, acc_tail, W1, b1, W2, b2):
    def body(a_ref, t_ref, w1_ref, b1_ref, w2_ref, b2_ref, o_ref):
        bagc = jnp.sum(a_ref[...] + t_ref[...], axis=1, keepdims=True)
        bagc = bagc * (1.0 / NTOK)
        h = lax.dot_general(
            bagc, w1_ref[...], (((0,), (0,)), ((), ())),
            preferred_element_type=jnp.float32,
        )
        h = jnp.maximum(h + b1_ref[...], 0.0)
        logits = jnp.dot(h, w2_ref[...], preferred_element_type=jnp.float32)
        logits = logits + b2_ref[...]
        m = jnp.max(logits, axis=-1, keepdims=True)
        lse = jnp.log(jnp.sum(jnp.exp(logits - m), axis=-1, keepdims=True)) + m
        o_ref[...] = logits - lse

    return pl.pallas_call(
        body,
        out_shape=jax.ShapeDtypeStruct((1, OUT), jnp.float32),
    )(acc_main, acc_tail, W1, b1.reshape(1, HID), W2, b2.reshape(1, OUT))


def kernel(input, embeddings, W1, b1, W2, b2):
    c0, c1 = _sc_counts(input.reshape(128, 128))
    c0 = c0.reshape(CROWS, CW)
    c1 = c1.reshape(CROWS, CW)
    table_t = embeddings.T
    acc_main = _tc_matvec_main(table_t, c0, c1)
    acc_tail = _tc_matvec_tail(table_t, c0, c1)
    return _tc_ffnn(acc_main, acc_tail, W1, b1, W2, b2)


# speedup vs baseline: 5.6968x; 1.2310x over previous
import functools

import jax
import jax.numpy as jnp
from jax import lax
from jax.experimental import pallas as pl
from jax.experimental.pallas import tpu as pltpu
from jax.experimental.pallas import tpu_sc as plsc

VOCAB = 1000000
EMB = 64
HID = 1024
OUT = 128
NTOK = 16384

LANES = 16

CW = 1664
CROWS = 608
CBIG = CROWS * CW
CPT = CBIG // 16
ZCH = CPT // 16

MBLK = 39936
NMAIN = 25
TAIL0 = NMAIN * MBLK
TAILW = CW


def _sc_counts(idx128):
    mesh = plsc.VectorSubcoreMesh(core_axis_name="c", subcore_axis_name="s")

    @functools.partial(
        pl.kernel,
        mesh=mesh,
        out_type=(
            jax.ShapeDtypeStruct((CBIG,), jnp.float32),
            jax.ShapeDtypeStruct((CBIG,), jnp.float32),
        ),
        scratch_types=[
            pltpu.VMEM((4, 128), jnp.int32),
            pltpu.VMEM((128,), jnp.float32),
            pltpu.VMEM((ZCH,), jnp.float32),
            pltpu.VMEM_SHARED((CBIG,), jnp.float32),
        ],
    )
    def k(idx_hbm, out0_hbm, out1_hbm, idx_v, ones_v, zeros_v, shared):
        s = lax.axis_index("s")
        c = lax.axis_index("c")
        g = s * 2 + c
        pltpu.sync_copy(idx_hbm.at[pl.ds(g * 4, 4)], idx_v)

        zero = jnp.zeros((LANES,), jnp.float32)

        def zv(r, carry):
            zeros_v[pl.ds(pl.multiple_of(r * LANES, LANES), LANES)] = zero
            return carry
        lax.fori_loop(0, ZCH // LANES, zv, 0)
        for l in range(8):
            ones_v[pl.ds(l * LANES, LANES)] = jnp.ones((LANES,), jnp.float32)

        base = pl.multiple_of(s * CPT, 8)
        for kk in range(16):
            pltpu.sync_copy(
                zeros_v, shared.at[pl.ds(base + kk * ZCH, ZCH)]
            )
        plsc.subcore_barrier()

        for j in range(4):
            pltpu.sync_copy(ones_v, shared.at[idx_v.at[j]], add=True)
        plsc.subcore_barrier()

        out = [out0_hbm, out1_hbm]
        for ci in range(2):
            @pl.when(c == ci)
            def _():
                pltpu.sync_copy(
                    shared.at[pl.ds(base, CPT)],
                    out[ci].at[pl.ds(base, CPT)],
                )

    return k(idx128)


def _tc_matvec_main(table_t, c0, c1):
    def body(t_ref, c0_ref, c1_ref, o_ref):
        cc = c0_ref[...] + c1_ref[...]
        tb = t_ref[...]
        acc = jnp.zeros((EMB, 128), jnp.float32)
        for gg in range(MBLK // 128):
            acc = acc + tb[:, gg * 128:(gg + 1) * 128] * cc[gg * 128:(gg + 1) * 128]

        @pl.when(pl.program_id(0) == 0)
        def _():
            o_ref[...] = jnp.zeros_like(o_ref)

        o_ref[...] += acc

    return pl.pallas_call(
        body,
        grid=(NMAIN,),
        in_specs=[
            pl.BlockSpec((EMB, MBLK), lambda i: (0, i)),
            pl.BlockSpec((MBLK,), lambda i: (i,)),
            pl.BlockSpec((MBLK,), lambda i: (i,)),
        ],
        out_specs=pl.BlockSpec((EMB, 128), lambda i: (0, 0)),
        out_shape=jax.ShapeDtypeStruct((EMB, 128), jnp.float32),
    )(table_t, c0, c1)


def _tc_ffnn_tail(acc_main, table_t, c0, c1, W1, b1, W2, b2):
    def body(a_ref, t_ref, c0_ref, c1_ref, w1_ref, b1_ref, w2_ref, b2_ref,
             o_ref):
        cc = c0_ref[pl.ds(0, TAILW)] + c1_ref[pl.ds(0, TAILW)]
        tb = t_ref[...]
        acc = a_ref[...]
        for gg in range(TAILW // 128):
            prod = tb[:, gg * 128:(gg + 1) * 128] * cc[gg * 128:(gg + 1) * 128]
            if (gg + 1) * 128 > VOCAB - TAIL0:
                valid = jax.lax.broadcasted_iota(jnp.int32, (1, 128), 1) < (
                    VOCAB - TAIL0 - gg * 128
                )
                prod = jnp.where(valid, prod, 0.0)
            acc = acc + prod

        bagc = jnp.sum(acc, axis=1, keepdims=True) * (1.0 / NTOK)
        h = lax.dot_general(
            bagc, w1_ref[...], (((0,), (0,)), ((), ())),
            preferred_element_type=jnp.float32,
        )
        h = jnp.maximum(h + b1_ref[...], 0.0)
        logits = jnp.dot(h, w2_ref[...], preferred_element_type=jnp.float32)
        logits = logits + b2_ref[...]
        m = jnp.max(logits, axis=-1, keepdims=True)
        lse = jnp.log(jnp.sum(jnp.exp(logits - m), axis=-1, keepdims=True)) + m
        o_ref[...] = logits - lse

    return pl.pallas_call(
        body,
        grid=(1,),
        in_specs=[
            pl.BlockSpec((EMB, 128), lambda i: (0, 0)),
            pl.BlockSpec((EMB, TAILW), lambda i: (0, TAIL0 // TAILW)),
            pl.BlockSpec((13312,), lambda i: (TAIL0 // 13312,)),
            pl.BlockSpec((13312,), lambda i: (TAIL0 // 13312,)),
            pl.BlockSpec((EMB, HID), lambda i: (0, 0)),
            pl.BlockSpec((1, HID), lambda i: (0, 0)),
            pl.BlockSpec((HID, OUT), lambda i: (0, 0)),
            pl.BlockSpec((1, OUT), lambda i: (0, 0)),
        ],
        out_specs=pl.BlockSpec((1, OUT), lambda i: (0, 0)),
        out_shape=jax.ShapeDtypeStruct((1, OUT), jnp.float32),
    )(acc_main, table_t, c0, c1, W1, b1.reshape(1, HID), W2,
      b2.reshape(1, OUT))


def kernel(input, embeddings, W1, b1, W2, b2):
    c0, c1 = _sc_counts(input.reshape(128, 128))
    table_t = embeddings.T
    acc_main = _tc_matvec_main(table_t, c0, c1)
    return _tc_ffnn_tail(acc_main, table_t, c0, c1, W1, b1, W2, b2)


# speedup vs baseline: 5.7395x; 1.0075x over previous
import functools

import jax
import jax.numpy as jnp
from jax import lax
from jax.experimental import pallas as pl
from jax.experimental.pallas import tpu as pltpu
from jax.experimental.pallas import tpu_sc as plsc

VOCAB = 1000000
EMB = 64
HID = 1024
OUT = 128
NTOK = 16384

LANES = 16

CW = 1664
CROWS = 608
CBIG = CROWS * CW
CPT = CBIG // 16
ZCH = CPT // 16

MBLK = 39936
NMAIN = 25
TAIL0 = NMAIN * MBLK
TAILW = CW


def _sc_counts(idx128):
    mesh = plsc.VectorSubcoreMesh(core_axis_name="c", subcore_axis_name="s")

    @functools.partial(
        pl.kernel,
        mesh=mesh,
        out_type=(
            jax.ShapeDtypeStruct((CBIG,), jnp.float32),
            jax.ShapeDtypeStruct((CBIG,), jnp.float32),
        ),
        scratch_types=[
            pltpu.VMEM((4, 128), jnp.int32),
            pltpu.VMEM((128,), jnp.float32),
            pltpu.VMEM((ZCH,), jnp.float32),
            pltpu.VMEM_SHARED((CBIG,), jnp.float32),
            pltpu.SemaphoreType.DMA,
        ],
    )
    def k(idx_hbm, out0_hbm, out1_hbm, idx_v, ones_v, zeros_v, shared, sem):
        s = lax.axis_index("s")
        c = lax.axis_index("c")
        g = s * 2 + c
        pltpu.sync_copy(idx_hbm.at[pl.ds(g * 4, 4)], idx_v)

        zero = jnp.zeros((LANES,), jnp.float32)

        def zv(r, carry):
            zeros_v[pl.ds(pl.multiple_of(r * LANES, LANES), LANES)] = zero
            return carry
        lax.fori_loop(0, ZCH // LANES, zv, 0)
        for l in range(8):
            ones_v[pl.ds(l * LANES, LANES)] = jnp.ones((LANES,), jnp.float32)

        base = pl.multiple_of(s * CPT, 8)
        zcopies = [
            pltpu.async_copy(
                zeros_v, shared.at[pl.ds(base + kk * ZCH, ZCH)], sem
            )
            for kk in range(16)
        ]
        for cp in zcopies:
            cp.wait()
        plsc.subcore_barrier()

        scopies = [
            pltpu.async_copy(ones_v, shared.at[idx_v.at[j]], sem, add=True)
            for j in range(4)
        ]
        for cp in scopies:
            cp.wait()
        plsc.subcore_barrier()

        out = [out0_hbm, out1_hbm]
        for ci in range(2):
            @pl.when(c == ci)
            def _():
                pltpu.sync_copy(
                    shared.at[pl.ds(base, CPT)],
                    out[ci].at[pl.ds(base, CPT)],
                )

    return k(idx128)


def _tc_matvec_main(table_t, c0, c1):
    def body(t_ref, c0_ref, c1_ref, o_ref):
        cc = c0_ref[...] + c1_ref[...]
        tb = t_ref[...]
        acc = jnp.zeros((EMB, 128), jnp.float32)
        for gg in range(MBLK // 128):
            acc = acc + tb[:, gg * 128:(gg + 1) * 128] * cc[gg * 128:(gg + 1) * 128]

        @pl.when(pl.program_id(0) == 0)
        def _():
            o_ref[...] = jnp.zeros_like(o_ref)

        o_ref[...] += acc

    return pl.pallas_call(
        body,
        grid=(NMAIN,),
        in_specs=[
            pl.BlockSpec((EMB, MBLK), lambda i: (0, i)),
            pl.BlockSpec((MBLK,), lambda i: (i,)),
            pl.BlockSpec((MBLK,), lambda i: (i,)),
        ],
        out_specs=pl.BlockSpec((EMB, 128), lambda i: (0, 0)),
        out_shape=jax.ShapeDtypeStruct((EMB, 128), jnp.float32),
    )(table_t, c0, c1)


def _tc_ffnn_tail(acc_main, table_t, c0, c1, W1, b1, W2, b2):
    def body(a_ref, t_ref, c0_ref, c1_ref, w1_ref, b1_ref, w2_ref, b2_ref,
             o_ref):
        cc = c0_ref[pl.ds(0, TAILW)] + c1_ref[pl.ds(0, TAILW)]
        tb = t_ref[...]
        acc = a_ref[...]
        for gg in range(TAILW // 128):
            prod = tb[:, gg * 128:(gg + 1) * 128] * cc[gg * 128:(gg + 1) * 128]
            if (gg + 1) * 128 > VOCAB - TAIL0:
                valid = jax.lax.broadcasted_iota(jnp.int32, (1, 128), 1) < (
                    VOCAB - TAIL0 - gg * 128
                )
                prod = jnp.where(valid, prod, 0.0)
            acc = acc + prod

        bagc = jnp.sum(acc, axis=1, keepdims=True) * (1.0 / NTOK)
        h = lax.dot_general(
            bagc, w1_ref[...], (((0,), (0,)), ((), ())),
            preferred_element_type=jnp.float32,
        )
        h = jnp.maximum(h + b1_ref[...], 0.0)
        logits = jnp.dot(h, w2_ref[...], preferred_element_type=jnp.float32)
        logits = logits + b2_ref[...]
        m = jnp.max(logits, axis=-1, keepdims=True)
        lse = jnp.log(jnp.sum(jnp.exp(logits - m), axis=-1, keepdims=True)) + m
        o_ref[...] = logits - lse

    return pl.pallas_call(
        body,
        grid=(1,),
        in_specs=[
            pl.BlockSpec((EMB, 128), lambda i: (0, 0)),
            pl.BlockSpec((EMB, TAILW), lambda i: (0, TAIL0 // TAILW)),
            pl.BlockSpec((13312,), lambda i: (TAIL0 // 13312,)),
            pl.BlockSpec((13312,), lambda i: (TAIL0 // 13312,)),
            pl.BlockSpec((EMB, HID), lambda i: (0, 0)),
            pl.BlockSpec((1, HID), lambda i: (0, 0)),
            pl.BlockSpec((HID, OUT), lambda i: (0, 0)),
            pl.BlockSpec((1, OUT), lambda i: (0, 0)),
        ],
        out_specs=pl.BlockSpec((1, OUT), lambda i: (0, 0)),
        out_shape=jax.ShapeDtypeStruct((1, OUT), jnp.float32),
    )(acc_main, table_t, c0, c1, W1, b1.reshape(1, HID), W2,
      b2.reshape(1, OUT))


def kernel(input, embeddings, W1, b1, W2, b2):
    c0, c1 = _sc_counts(input.reshape(128, 128))
    table_t = embeddings.T
    acc_main = _tc_matvec_main(table_t, c0, c1)
    return _tc_ffnn_tail(acc_main, table_t, c0, c1, W1, b1, W2, b2)
